# Initial kernel scaffold; baseline (speedup 1.0000x reference)
import jax
import jax.numpy as jnp
from jax.experimental import pallas as pl


def kernel(x, b1x1_w, b1x1_gamma, b1x1_beta, b1x1_mean, b1x1_var, b5x5_1_w, b5x5_1_gamma, b5x5_1_beta, b5x5_1_mean, b5x5_1_var, b5x5_2_w, b5x5_2_gamma, b5x5_2_beta, b5x5_2_mean, b5x5_2_var, b3x3_1_w, b3x3_1_gamma, b3x3_1_beta, b3x3_1_mean, b3x3_1_var, b3x3_2_w, b3x3_2_gamma, b3x3_2_beta, b3x3_2_mean, b3x3_2_var, b3x3_3_w, b3x3_3_gamma, b3x3_3_beta, b3x3_3_mean, b3x3_3_var, bpool_w, bpool_gamma, bpool_beta, bpool_mean, bpool_var):
    raise NotImplementedError("write your pallas kernel here")


# speedup vs baseline: 1.6694x; 1.6694x over previous
from functools import partial

import numpy as np
import jax
import jax.numpy as jnp
from jax.experimental import pallas as pl
from jax.experimental.pallas import tpu as pltpu

_BN_EPS = 1e-3
_VMEM_LIMIT = 48 * 1024 * 1024


def _rup(a, b):
    return ((a + b - 1) // b) * b


def _fold_bn(w, gamma, beta, mean, var):
    scale = gamma / jnp.sqrt(var + _BN_EPS)
    return w * scale[:, None, None, None], beta - mean * scale


def _interior_mask(H, W, Wp, LW, r):
    m = np.zeros((1, LW), np.float32)
    p = np.arange((H + 2 * r) * Wp)
    y, x = p // Wp, p % Wp
    ok = (y >= r) & (y < H + r) & (x >= r) & (x < W + r)
    m[0, : p.size] = ok.astype(np.float32)
    return jnp.asarray(m)


def _body(x_ref, ws_ref, bm_ref, w3a_ref, b3a_ref, w5_ref, b5_ref,
          w3b_ref, b3b_ref, bp_ref, mask_ref, o_ref,
          t5p, t3p, upp, t3bp, stk,
          *, H, W, Wp, MG, LW, c1, c5i, c3i, pf, c5, c3m, c3):
    r = 2
    nm = c1 + c5i + c3i

    xb = x_ref[0].astype(jnp.bfloat16)
    y = jnp.dot(ws_ref[...], xb, preferred_element_type=jnp.float32)
    ym = jnp.maximum(y[:nm] + bm_ref[...], 0.0)
    o_ref[0, 0:c1, :] = ym[0:c1]

    t5p[...] = jnp.zeros_like(t5p)
    t3p[...] = jnp.zeros_like(t3p)
    upp[...] = jnp.zeros_like(upp)
    t5c = ym[c1:c1 + c5i].astype(jnp.bfloat16)
    t3c = ym[c1 + c5i:nm].astype(jnp.bfloat16)
    upc = y[nm:]
    for row in range(H):
        s = row * W
        d = MG + (row + r) * Wp + r
        t5p[:, d:d + W] = t5c[:, s:s + W]
        t3p[:, d:d + W] = t3c[:, s:s + W]
        upp[:, d:d + W] = upc[:, s:s + W]

    for t in range(9):
        ky, kx = divmod(t, 3)
        off = MG + (ky - 1) * Wp + (kx - 1)
        stk[t * c3i:(t + 1) * c3i, :] = t3p[:, off:off + LW]
    y3 = jnp.dot(w3a_ref[...], stk[0:9 * c3i],
                 preferred_element_type=jnp.float32)
    t3bv = (jnp.maximum(y3 + b3a_ref[...], 0.0) * mask_ref[...]
            ).astype(jnp.bfloat16)
    t3bp[:, 0:MG] = jnp.zeros((c3m, MG), jnp.bfloat16)
    t3bp[:, MG:MG + LW] = t3bv
    t3bp[:, MG + LW:] = jnp.zeros((c3m, t3bp.shape[1] - MG - LW), jnp.bfloat16)

    for t in range(9):
        ky, kx = divmod(t, 3)
        off = MG + (ky - 1) * Wp + (kx - 1)
        stk[t * c3m:(t + 1) * c3m, :] = t3bp[:, off:off + LW]
    y3b = jnp.maximum(
        jnp.dot(w3b_ref[...], stk[0:9 * c3m],
                preferred_element_type=jnp.float32) + b3b_ref[...], 0.0)

    for t in range(25):
        ky, kx = divmod(t, 5)
        off = MG + (ky - 2) * Wp + (kx - 2)
        stk[t * c5i:(t + 1) * c5i, :] = t5p[:, off:off + LW]
    y5 = jnp.maximum(
        jnp.dot(w5_ref[...], stk[0:25 * c5i],
                preferred_element_type=jnp.float32) + b5_ref[...], 0.0)

    pacc = jnp.zeros((pf, LW), jnp.float32)
    for t in range(9):
        ky, kx = divmod(t, 3)
        off = MG + (ky - 1) * Wp + (kx - 1)
        pacc = pacc + upp[:, off:off + LW]
    yp = jnp.maximum(pacc * (1.0 / 9.0) + bp_ref[...], 0.0)

    for row in range(H):
        s = (row + r) * Wp + r
        d = row * W
        o_ref[0, c1:c1 + c5, d:d + W] = y5[:, s:s + W]
        o_ref[0, c1 + c5:c1 + c5 + c3, d:d + W] = y3b[:, s:s + W]
        o_ref[0, c1 + c5 + c3:, d:d + W] = yp[:, s:s + W]


def kernel(x, b1x1_w, b1x1_gamma, b1x1_beta, b1x1_mean, b1x1_var,
           b5x5_1_w, b5x5_1_gamma, b5x5_1_beta, b5x5_1_mean, b5x5_1_var,
           b5x5_2_w, b5x5_2_gamma, b5x5_2_beta, b5x5_2_mean, b5x5_2_var,
           b3x3_1_w, b3x3_1_gamma, b3x3_1_beta, b3x3_1_mean, b3x3_1_var,
           b3x3_2_w, b3x3_2_gamma, b3x3_2_beta, b3x3_2_mean, b3x3_2_var,
           b3x3_3_w, b3x3_3_gamma, b3x3_3_beta, b3x3_3_mean, b3x3_3_var,
           bpool_w, bpool_gamma, bpool_beta, bpool_mean, bpool_var):
    B, Cin, H, W = map(int, x.shape)
    HW = H * W
    r = 2
    Wp, Hp = W + 2 * r, H + 2 * r
    Lp = Hp * Wp
    LW = _rup(Lp, 128)
    maxoff = r * Wp + r
    MG = _rup(maxoff, 128)
    LT = _rup(MG + maxoff + LW, 128)

    w1, b1 = _fold_bn(b1x1_w, b1x1_gamma, b1x1_beta, b1x1_mean, b1x1_var)
    w51, b51 = _fold_bn(b5x5_1_w, b5x5_1_gamma, b5x5_1_beta, b5x5_1_mean,
                        b5x5_1_var)
    w52, b52 = _fold_bn(b5x5_2_w, b5x5_2_gamma, b5x5_2_beta, b5x5_2_mean,
                        b5x5_2_var)
    w31, b31 = _fold_bn(b3x3_1_w, b3x3_1_gamma, b3x3_1_beta, b3x3_1_mean,
                        b3x3_1_var)
    w32, b32 = _fold_bn(b3x3_2_w, b3x3_2_gamma, b3x3_2_beta, b3x3_2_mean,
                        b3x3_2_var)
    w33, b33 = _fold_bn(b3x3_3_w, b3x3_3_gamma, b3x3_3_beta, b3x3_3_mean,
                        b3x3_3_var)
    wp, bp = _fold_bn(bpool_w, bpool_gamma, bpool_beta, bpool_mean, bpool_var)

    c1, c5i, c3i, pf = (w1.shape[0], w51.shape[0], w31.shape[0], wp.shape[0])
    c5, c3m, c3 = w52.shape[0], w32.shape[0], w33.shape[0]
    nm = c1 + c5i + c3i
    ctot = c1 + c5 + c3 + pf

    bf = jnp.bfloat16
    ws = jnp.concatenate(
        [w1[:, :, 0, 0], w51[:, :, 0, 0], w31[:, :, 0, 0], wp[:, :, 0, 0]],
        0).astype(bf)
    bm = jnp.concatenate([b1, b51, b31], 0).reshape(nm, 1)
    w5s = w52.transpose(0, 2, 3, 1).reshape(c5, 25 * c5i).astype(bf)
    w3as = w32.transpose(0, 2, 3, 1).reshape(c3m, 9 * c3i).astype(bf)
    w3bs = w33.transpose(0, 2, 3, 1).reshape(c3, 9 * c3m).astype(bf)
    mask = _interior_mask(H, W, Wp, LW, r)

    kst = max(25 * c5i, 9 * c3i, 9 * c3m)

    def const(shape):
        n = len(shape)
        return pl.BlockSpec(shape, lambda b, _n=n: (0,) * _n)

    out = pl.pallas_call(
        partial(_body, H=H, W=W, Wp=Wp, MG=MG, LW=LW, c1=c1, c5i=c5i,
                c3i=c3i, pf=pf, c5=c5, c3m=c3m, c3=c3),
        out_shape=jax.ShapeDtypeStruct((B, ctot, HW), jnp.float32),
        grid=(B,),
        in_specs=[
            pl.BlockSpec((1, Cin, HW), lambda b: (b, 0, 0)),
            const((nm + pf, Cin)),
            const((nm, 1)),
            const((c3m, 9 * c3i)),
            const((c3m, 1)),
            const((c5, 25 * c5i)),
            const((c5, 1)),
            const((c3, 9 * c3m)),
            const((c3, 1)),
            const((pf, 1)),
            const((1, LW)),
        ],
        out_specs=pl.BlockSpec((1, ctot, HW), lambda b: (b, 0, 0)),
        scratch_shapes=[
            pltpu.VMEM((c5i, LT), bf),
            pltpu.VMEM((c3i, LT), bf),
            pltpu.VMEM((pf, LT), jnp.float32),
            pltpu.VMEM((c3m, LT), bf),
            pltpu.VMEM((kst, LW), bf),
        ],
        compiler_params=pltpu.CompilerParams(
            dimension_semantics=("parallel",),
            vmem_limit_bytes=_VMEM_LIMIT),
    )(x.reshape(B, Cin, HW), ws, bm, w3as, b32.reshape(c3m, 1),
      w5s, b52.reshape(c5, 1), w3bs, b33.reshape(c3, 1),
      bp.reshape(pf, 1), mask)

    return out.reshape(B, ctot, H, W)
